# baseline (speedup 1.0000x reference)
import functools

import jax
import jax.numpy as jnp
from jax import lax
from jax.experimental import pallas as pl
from jax.experimental.pallas import tpu as pltpu
from jax.experimental.pallas import tpu_sc as plsc

N = 10000
E = 320000
NTILES = 16
CHUNK = 128
EPAD = 327680
RACC = 10240
FW = 128


G = 16


def _spmm_body(kch, h_hbm, src_hbm, dst_hbm, init_hbm, out_hbm,
               src_v, dst_v, buf_a, buf_b, init_v, acc,
               sem_a, sem_b, sem_sa, sem_sb, sem_i):
    c = lax.axis_index("c")
    s = lax.axis_index("s")

    pltpu.sync_copy(init_hbm.at[c], init_v)

    def init(j, carry):
        pltpu.async_copy(init_v, acc.at[pl.ds(s * 640 + j * 16, 16)], sem_sa)
        return carry

    lax.fori_loop(0, 40, init, 0)

    def init_wait(j, carry):
        pltpu.make_async_copy(init_v, acc.at[pl.ds(0, 16)], sem_sa).wait()
        return carry

    lax.fori_loop(0, 40, init_wait, 0)
    plsc.subcore_barrier()

    def wait_g(buf, sem):
        pltpu.make_async_copy(h_hbm.at[pl.ds(0, CHUNK)], buf, sem).wait()

    def wait_i():
        pltpu.make_async_copy(src_hbm.at[c, s, pl.ds(0, G)], src_v.at[0], sem_i).wait()

    ngroups = kch // G

    pltpu.async_copy(src_hbm.at[c, s, pl.ds(0, G)], src_v.at[0], sem_i)
    pltpu.async_copy(dst_hbm.at[c, s, pl.ds(0, G)], dst_v.at[0], sem_i)
    wait_i()
    wait_i()
    pltpu.async_copy(h_hbm.at[src_v.at[0, 0]], buf_a, sem_a)
    pltpu.async_copy(h_hbm.at[src_v.at[0, 1]], buf_b, sem_b)
    if ngroups > 1:
        pltpu.async_copy(src_hbm.at[c, s, pl.ds(G, G)], src_v.at[1], sem_i)
        pltpu.async_copy(dst_hbm.at[c, s, pl.ds(G, G)], dst_v.at[1], sem_i)

    def group(g, carry):
        p = lax.rem(g, 2)

        def pair(j, carry):
            k0 = 2 * j
            wait_g(buf_a, sem_a)
            pltpu.sync_copy(buf_a, acc.at[dst_v.at[p, k0]], add=True)
            pltpu.async_copy(h_hbm.at[src_v.at[p, k0 + 2]], buf_a, sem_a)
            wait_g(buf_b, sem_b)
            pltpu.sync_copy(buf_b, acc.at[dst_v.at[p, k0 + 1]], add=True)
            pltpu.async_copy(h_hbm.at[src_v.at[p, k0 + 3]], buf_b, sem_b)
            return carry

        lax.fori_loop(0, G // 2 - 1, pair, 0)
        wait_g(buf_a, sem_a)
        pltpu.sync_copy(buf_a, acc.at[dst_v.at[p, G - 2]], add=True)

        @pl.when(g + 1 < ngroups)
        def _():
            wait_i()
            wait_i()
            pltpu.async_copy(h_hbm.at[src_v.at[1 - p, 0]], buf_a, sem_a)

        wait_g(buf_b, sem_b)
        pltpu.sync_copy(buf_b, acc.at[dst_v.at[p, G - 1]], add=True)

        @pl.when(g + 1 < ngroups)
        def _():
            pltpu.async_copy(h_hbm.at[src_v.at[1 - p, 1]], buf_b, sem_b)

            @pl.when(g + 2 < ngroups)
            def _():
                pltpu.async_copy(src_hbm.at[c, s, pl.ds((g + 2) * G, G)],
                                 src_v.at[p], sem_i)
                pltpu.async_copy(dst_hbm.at[c, s, pl.ds((g + 2) * G, G)],
                                 dst_v.at[p], sem_i)

        return carry

    lax.fori_loop(0, ngroups, group, 0)
    plsc.subcore_barrier()

    @pl.when(s < 15)
    def _():
        pltpu.sync_copy(acc.at[pl.ds(s * 640, 640)], out_hbm.at[c, pl.ds(s * 640, 640)])

    @pl.when(s == 15)
    def _():
        pltpu.sync_copy(acc.at[pl.ds(9600, 400)], out_hbm.at[c, pl.ds(9600, 400)])


@functools.cache
def _make_spmm(kch: int):
    mesh = plsc.VectorSubcoreMesh(core_axis_name="c", subcore_axis_name="s")
    return pl.kernel(
        functools.partial(_spmm_body, kch),
        out_type=jax.ShapeDtypeStruct((2, N, FW), jnp.float32),
        mesh=mesh,
        scratch_types=[
            pltpu.VMEM((2, G, CHUNK), jnp.int32),
            pltpu.VMEM((2, G, CHUNK), jnp.int32),
            pltpu.VMEM((CHUNK, FW), jnp.float32),
            pltpu.VMEM((CHUNK, FW), jnp.float32),
            pltpu.VMEM((16, FW), jnp.float32),
            pltpu.VMEM_SHARED((RACC, FW), jnp.float32),
            pltpu.SemaphoreType.DMA,
            pltpu.SemaphoreType.DMA,
            pltpu.SemaphoreType.DMA,
            pltpu.SemaphoreType.DMA,
            pltpu.SemaphoreType.DMA,
        ],
        name=f"sc_spmm_k{kch}",
    )


def _mm_body(relu, split_out, post, h_ref, w_ref, b_ref, *rest):
    o_ref = rest[-1]
    bf = jnp.bfloat16
    res = (jnp.dot(h_ref[0].astype(bf), w_ref[0].astype(bf),
                   preferred_element_type=jnp.float32)
           + jnp.dot(h_ref[1].astype(bf), w_ref[1].astype(bf),
                     preferred_element_type=jnp.float32)
           + b_ref[...])
    if relu:
        res = jnp.maximum(res, 0.0)
    if post:
        wp_ref = rest[0]
        res = jnp.dot(res.astype(bf), wp_ref[...].astype(bf),
                      preferred_element_type=jnp.float32)
        bm = res.shape[0]
        o_ref[...] = jnp.concatenate(
            [res, jnp.zeros((bm, FW - res.shape[1]), jnp.float32)], axis=1)
    elif split_out:
        o_ref[0] = res[:, :FW]
        o_ref[1] = res[:, FW:]
    else:
        o_ref[...] = res


def _mm2(a, wst, b, relu, split_out, w_post=None):
    fo = wst.shape[2]
    bm = 1000
    in_specs = [
        pl.BlockSpec((2, bm, FW), lambda i: (0, i, 0)),
        pl.BlockSpec((2, FW, fo), lambda i: (0, 0, 0)),
        pl.BlockSpec((1, fo), lambda i: (0, 0)),
    ]
    args = [a, wst, b.reshape(1, fo)]
    if w_post is not None:
        out_shape = jax.ShapeDtypeStruct((N, FW), jnp.float32)
        out_spec = pl.BlockSpec((bm, FW), lambda i: (i, 0))
        in_specs.append(pl.BlockSpec(w_post.shape, lambda i: (0, 0)))
        args.append(w_post)
    elif split_out:
        out_shape = jax.ShapeDtypeStruct((2, N, FW), jnp.float32)
        out_spec = pl.BlockSpec((2, bm, FW), lambda i: (0, i, 0))
    else:
        out_shape = jax.ShapeDtypeStruct((N, fo), jnp.float32)
        out_spec = pl.BlockSpec((bm, fo), lambda i: (i, 0))
    return pl.pallas_call(
        functools.partial(_mm_body, relu, split_out, w_post is not None),
        grid=(N // bm,),
        in_specs=in_specs,
        out_specs=out_spec,
        out_shape=out_shape,
    )(*args)


def _merge_body(fo, a_ref, b_ref, o_ref):
    o_ref[...] = (a_ref[0] + a_ref[1])[:, :fo] + b_ref[...]


def _merge(a, b4):
    fo = b4.shape[0]
    bm = 1000
    return pl.pallas_call(
        functools.partial(_merge_body, fo),
        grid=(N // bm,),
        in_specs=[
            pl.BlockSpec((2, bm, FW), lambda i: (0, i, 0)),
            pl.BlockSpec((1, fo), lambda i: (0, 0)),
        ],
        out_specs=pl.BlockSpec((bm, fo), lambda i: (i, 0)),
        out_shape=jax.ShapeDtypeStruct((N, fo), jnp.float32),
    )(a, b4.reshape(1, fo))


def kernel(x, edge_index, W0, W1, W2, W3, W4, b0, b1, b2, b3, b4):
    dst = edge_index[0].astype(jnp.int32)
    src = edge_index[1].astype(jnp.int32)
    pad = EPAD - E
    pad_idx = jnp.arange(pad, dtype=jnp.int32)
    dst_p = jnp.concatenate([dst, N + pad_idx % (RACC - N)])
    src_p = jnp.concatenate([src, pad_idx % N])
    src_r = src_p.reshape(NTILES, 160, CHUNK)
    dst_r = dst_p.reshape(NTILES, 160, CHUNK)
    src_fs = jnp.stack([src_r, src_r + N])
    dst_fs = jnp.stack([dst_r, dst_r])
    src_es = src_p.reshape(2, NTILES, 80, CHUNK)
    dst_es = dst_p.reshape(2, NTILES, 80, CHUNK)

    z_init = jnp.zeros((2, 16, FW), jnp.float32)
    spmm_fs = _make_spmm(160)
    spmm_es = _make_spmm(80)

    a = spmm_es(x, src_es, dst_es, z_init)
    h = _mm2(a, jnp.stack([W0, W0]), b0, relu=True, split_out=True)
    for W, b in ((W1, b1), (W2, b2)):
        a = spmm_fs(h.reshape(2 * N, FW), src_fs, dst_fs, z_init)
        h = _mm2(a, jnp.stack([W[:FW], W[FW:]]), b, relu=True, split_out=True)
    a = spmm_fs(h.reshape(2 * N, FW), src_fs, dst_fs, z_init)
    p = _mm2(a, jnp.stack([W3[:FW], W3[FW:]]), b3, relu=True, split_out=False,
             w_post=W4)
    a = spmm_es(p, src_es, dst_es, z_init)
    return _merge(a, b4)


# speedup vs baseline: 1.0124x; 1.0124x over previous
import functools

import jax
import jax.numpy as jnp
from jax import lax
from jax.experimental import pallas as pl
from jax.experimental.pallas import tpu as pltpu
from jax.experimental.pallas import tpu_sc as plsc

N = 10000
E = 320000
NTILES = 16
CHUNK = 128
EPAD = 327680
RACC = 10240
FW = 128


G = 16


def _spmm_body(kch, h_hbm, src_hbm, dst_hbm, init_hbm, out_hbm,
               src_v, dst_v, buf_a, buf_b, init_v, acc,
               sem_a, sem_b, sem_sa, sem_sb, sem_i):
    c = lax.axis_index("c")
    s = lax.axis_index("s")

    pltpu.sync_copy(init_hbm.at[c], init_v)

    def init(j, carry):
        pltpu.async_copy(init_v, acc.at[pl.ds(s * 640 + j * 16, 16)], sem_sa)
        return carry

    lax.fori_loop(0, 40, init, 0)

    def init_wait(j, carry):
        pltpu.make_async_copy(init_v, acc.at[pl.ds(0, 16)], sem_sa).wait()
        return carry

    lax.fori_loop(0, 40, init_wait, 0)
    plsc.subcore_barrier()

    def wait_g(buf, sem):
        pltpu.make_async_copy(h_hbm.at[pl.ds(0, CHUNK)], buf, sem).wait()

    def wait_i():
        pltpu.make_async_copy(src_hbm.at[c, s, pl.ds(0, G)], src_v.at[0], sem_i).wait()

    ngroups = kch // G

    pltpu.async_copy(src_hbm.at[c, s, pl.ds(0, G)], src_v.at[0], sem_i)
    pltpu.async_copy(dst_hbm.at[c, s, pl.ds(0, G)], dst_v.at[0], sem_i)
    wait_i()
    wait_i()
    pltpu.async_copy(h_hbm.at[src_v.at[0, 0]], buf_a, sem_a)
    pltpu.async_copy(h_hbm.at[src_v.at[0, 1]], buf_b, sem_b)
    if ngroups > 1:
        pltpu.async_copy(src_hbm.at[c, s, pl.ds(G, G)], src_v.at[1], sem_i)
        pltpu.async_copy(dst_hbm.at[c, s, pl.ds(G, G)], dst_v.at[1], sem_i)

    def group(g, carry):
        p = lax.rem(g, 2)

        def pair(j, carry):
            k0 = 2 * j
            wait_g(buf_a, sem_a)
            pltpu.sync_copy(buf_a, acc.at[dst_v.at[p, k0]], add=True)
            pltpu.async_copy(h_hbm.at[src_v.at[p, k0 + 2]], buf_a, sem_a)
            wait_g(buf_b, sem_b)
            pltpu.sync_copy(buf_b, acc.at[dst_v.at[p, k0 + 1]], add=True)
            pltpu.async_copy(h_hbm.at[src_v.at[p, k0 + 3]], buf_b, sem_b)
            return carry

        lax.fori_loop(0, G // 2 - 1, pair, 0)
        wait_g(buf_a, sem_a)
        pltpu.sync_copy(buf_a, acc.at[dst_v.at[p, G - 2]], add=True)

        @pl.when(g + 1 < ngroups)
        def _():
            wait_i()
            wait_i()
            pltpu.async_copy(h_hbm.at[src_v.at[1 - p, 0]], buf_a, sem_a)

        wait_g(buf_b, sem_b)
        pltpu.sync_copy(buf_b, acc.at[dst_v.at[p, G - 1]], add=True)

        @pl.when(g + 1 < ngroups)
        def _():
            pltpu.async_copy(h_hbm.at[src_v.at[1 - p, 1]], buf_b, sem_b)

            @pl.when(g + 2 < ngroups)
            def _():
                pltpu.async_copy(src_hbm.at[c, s, pl.ds((g + 2) * G, G)],
                                 src_v.at[p], sem_i)
                pltpu.async_copy(dst_hbm.at[c, s, pl.ds((g + 2) * G, G)],
                                 dst_v.at[p], sem_i)

        return carry

    lax.fori_loop(0, ngroups, group, 0)
    plsc.subcore_barrier()

    @pl.when(s < 15)
    def _():
        pltpu.sync_copy(acc.at[pl.ds(s * 640, 640)], out_hbm.at[c, pl.ds(s * 640, 640)])

    @pl.when(s == 15)
    def _():
        pltpu.sync_copy(acc.at[pl.ds(9600, 400)], out_hbm.at[c, pl.ds(9600, 400)])


@functools.cache
def _make_spmm(kch: int):
    mesh = plsc.VectorSubcoreMesh(core_axis_name="c", subcore_axis_name="s")
    return pl.kernel(
        functools.partial(_spmm_body, kch),
        out_type=jax.ShapeDtypeStruct((2, N, FW), jnp.float32),
        mesh=mesh,
        scratch_types=[
            pltpu.VMEM((2, G, CHUNK), jnp.int32),
            pltpu.VMEM((2, G, CHUNK), jnp.int32),
            pltpu.VMEM((CHUNK, FW), jnp.float32),
            pltpu.VMEM((CHUNK, FW), jnp.float32),
            pltpu.VMEM((16, FW), jnp.float32),
            pltpu.VMEM_SHARED((RACC, FW), jnp.float32),
            pltpu.SemaphoreType.DMA,
            pltpu.SemaphoreType.DMA,
            pltpu.SemaphoreType.DMA,
            pltpu.SemaphoreType.DMA,
            pltpu.SemaphoreType.DMA,
        ],
        name=f"sc_spmm_k{kch}",
    )


def _mm_body(relu, split_out, post, h_ref, w_ref, b_ref, *rest):
    o_ref = rest[-1]
    res = (jnp.dot(h_ref[0], w_ref[0], preferred_element_type=jnp.float32)
           + jnp.dot(h_ref[1], w_ref[1], preferred_element_type=jnp.float32)
           + b_ref[...])
    if relu:
        res = jnp.maximum(res, 0.0)
    if post:
        wp_ref = rest[0]
        res = jnp.dot(res, wp_ref[...], preferred_element_type=jnp.float32)
        bm = res.shape[0]
        o_ref[...] = jnp.concatenate(
            [res, jnp.zeros((bm, FW - res.shape[1]), jnp.float32)], axis=1)
    elif split_out:
        o_ref[0] = res[:, :FW]
        o_ref[1] = res[:, FW:]
    else:
        o_ref[...] = res


def _mm2(a, wst, b, relu, split_out, w_post=None):
    fo = wst.shape[2]
    bm = 2000
    in_specs = [
        pl.BlockSpec((2, bm, FW), lambda i: (0, i, 0)),
        pl.BlockSpec((2, FW, fo), lambda i: (0, 0, 0)),
        pl.BlockSpec((1, fo), lambda i: (0, 0)),
    ]
    args = [a, wst, b.reshape(1, fo)]
    if w_post is not None:
        out_shape = jax.ShapeDtypeStruct((N, FW), jnp.float32)
        out_spec = pl.BlockSpec((bm, FW), lambda i: (i, 0))
        in_specs.append(pl.BlockSpec(w_post.shape, lambda i: (0, 0)))
        args.append(w_post)
    elif split_out:
        out_shape = jax.ShapeDtypeStruct((2, N, FW), jnp.float32)
        out_spec = pl.BlockSpec((2, bm, FW), lambda i: (0, i, 0))
    else:
        out_shape = jax.ShapeDtypeStruct((N, fo), jnp.float32)
        out_spec = pl.BlockSpec((bm, fo), lambda i: (i, 0))
    return pl.pallas_call(
        functools.partial(_mm_body, relu, split_out, w_post is not None),
        grid=(N // bm,),
        in_specs=in_specs,
        out_specs=out_spec,
        out_shape=out_shape,
    )(*args)


def _merge_body(fo, a_ref, b_ref, o_ref):
    o_ref[...] = (a_ref[0] + a_ref[1])[:, :fo] + b_ref[...]


def _merge(a, b4):
    fo = b4.shape[0]
    bm = 2000
    return pl.pallas_call(
        functools.partial(_merge_body, fo),
        grid=(N // bm,),
        in_specs=[
            pl.BlockSpec((2, bm, FW), lambda i: (0, i, 0)),
            pl.BlockSpec((1, fo), lambda i: (0, 0)),
        ],
        out_specs=pl.BlockSpec((bm, fo), lambda i: (i, 0)),
        out_shape=jax.ShapeDtypeStruct((N, fo), jnp.float32),
    )(a, b4.reshape(1, fo))


def kernel(x, edge_index, W0, W1, W2, W3, W4, b0, b1, b2, b3, b4):
    dst = edge_index[0].astype(jnp.int32)
    src = edge_index[1].astype(jnp.int32)
    pad = EPAD - E
    pad_idx = jnp.arange(pad, dtype=jnp.int32)
    dst_p = jnp.concatenate([dst, N + pad_idx % (RACC - N)])
    src_p = jnp.concatenate([src, pad_idx % N])
    src_r = src_p.reshape(NTILES, 160, CHUNK)
    dst_r = dst_p.reshape(NTILES, 160, CHUNK)
    src_fs = jnp.stack([src_r, src_r + N])
    dst_fs = jnp.stack([dst_r, dst_r])
    src_es = src_p.reshape(2, NTILES, 80, CHUNK)
    dst_es = dst_p.reshape(2, NTILES, 80, CHUNK)

    z_init = jnp.zeros((2, 16, FW), jnp.float32)
    spmm_fs = _make_spmm(160)
    spmm_es = _make_spmm(80)

    a = spmm_es(x, src_es, dst_es, z_init)
    h = _mm2(a, jnp.stack([W0, W0]), b0, relu=True, split_out=True)
    for W, b in ((W1, b1), (W2, b2)):
        a = spmm_fs(h.reshape(2 * N, FW), src_fs, dst_fs, z_init)
        h = _mm2(a, jnp.stack([W[:FW], W[FW:]]), b, relu=True, split_out=True)
    a = spmm_fs(h.reshape(2 * N, FW), src_fs, dst_fs, z_init)
    p = _mm2(a, jnp.stack([W3[:FW], W3[FW:]]), b3, relu=True, split_out=False,
             w_post=W4)
    a = spmm_es(p, src_es, dst_es, z_init)
    return _merge(a, b4)


# speedup vs baseline: 1.0305x; 1.0178x over previous
import functools

import jax
import jax.numpy as jnp
from jax import lax
from jax.experimental import pallas as pl
from jax.experimental.pallas import tpu as pltpu
from jax.experimental.pallas import tpu_sc as plsc

N = 10000
E = 320000
NTILES = 16
CHUNK = 128
EPAD = 327680
RACC = 10240
FW = 128


G = 16


def _spmm_body(kch, h_hbm, src_hbm, dst_hbm, init_hbm, out_hbm,
               src_v, dst_v, buf_a, buf_b, init_v, acc,
               sem_a, sem_b, sem_sa, sem_sb, sem_i):
    c = lax.axis_index("c")
    s = lax.axis_index("s")

    def wait_g(buf, sem):
        pltpu.make_async_copy(h_hbm.at[pl.ds(0, CHUNK)], buf, sem).wait()

    def wait_i():
        pltpu.make_async_copy(src_hbm.at[c, s, pl.ds(0, G)], src_v.at[0], sem_i).wait()

    ngroups = kch // G

    pltpu.async_copy(src_hbm.at[c, s, pl.ds(0, G)], src_v.at[0], sem_i)
    pltpu.async_copy(dst_hbm.at[c, s, pl.ds(0, G)], dst_v.at[0], sem_i)
    pltpu.sync_copy(init_hbm.at[c], init_v)
    wait_i()
    wait_i()
    pltpu.async_copy(h_hbm.at[src_v.at[0, 0]], buf_a, sem_a)
    pltpu.async_copy(h_hbm.at[src_v.at[0, 1]], buf_b, sem_b)
    if ngroups > 1:
        pltpu.async_copy(src_hbm.at[c, s, pl.ds(G, G)], src_v.at[1], sem_i)
        pltpu.async_copy(dst_hbm.at[c, s, pl.ds(G, G)], dst_v.at[1], sem_i)

    def init(j, carry):
        pltpu.async_copy(init_v, acc.at[pl.ds(s * 640 + j * 16, 16)], sem_sa)
        return carry

    lax.fori_loop(0, 40, init, 0)

    def init_wait(j, carry):
        pltpu.make_async_copy(init_v, acc.at[pl.ds(0, 16)], sem_sa).wait()
        return carry

    lax.fori_loop(0, 40, init_wait, 0)
    plsc.subcore_barrier()

    def group(g, carry):
        p = lax.rem(g, 2)

        def pair(j, carry):
            k0 = 2 * j
            wait_g(buf_a, sem_a)
            pltpu.sync_copy(buf_a, acc.at[dst_v.at[p, k0]], add=True)
            pltpu.async_copy(h_hbm.at[src_v.at[p, k0 + 2]], buf_a, sem_a)
            wait_g(buf_b, sem_b)
            pltpu.sync_copy(buf_b, acc.at[dst_v.at[p, k0 + 1]], add=True)
            pltpu.async_copy(h_hbm.at[src_v.at[p, k0 + 3]], buf_b, sem_b)
            return carry

        lax.fori_loop(0, G // 2 - 1, pair, 0)
        wait_g(buf_a, sem_a)
        pltpu.sync_copy(buf_a, acc.at[dst_v.at[p, G - 2]], add=True)

        @pl.when(g + 1 < ngroups)
        def _():
            wait_i()
            wait_i()
            pltpu.async_copy(h_hbm.at[src_v.at[1 - p, 0]], buf_a, sem_a)

        wait_g(buf_b, sem_b)
        pltpu.sync_copy(buf_b, acc.at[dst_v.at[p, G - 1]], add=True)

        @pl.when(g + 1 < ngroups)
        def _():
            pltpu.async_copy(h_hbm.at[src_v.at[1 - p, 1]], buf_b, sem_b)

            @pl.when(g + 2 < ngroups)
            def _():
                pltpu.async_copy(src_hbm.at[c, s, pl.ds((g + 2) * G, G)],
                                 src_v.at[p], sem_i)
                pltpu.async_copy(dst_hbm.at[c, s, pl.ds((g + 2) * G, G)],
                                 dst_v.at[p], sem_i)

        return carry

    lax.fori_loop(0, ngroups, group, 0)
    plsc.subcore_barrier()

    @pl.when(s < 15)
    def _():
        pltpu.sync_copy(acc.at[pl.ds(s * 640, 640)], out_hbm.at[c, pl.ds(s * 640, 640)])

    @pl.when(s == 15)
    def _():
        pltpu.sync_copy(acc.at[pl.ds(9600, 400)], out_hbm.at[c, pl.ds(9600, 400)])


@functools.cache
def _make_spmm(kch: int):
    mesh = plsc.VectorSubcoreMesh(core_axis_name="c", subcore_axis_name="s")
    return pl.kernel(
        functools.partial(_spmm_body, kch),
        out_type=jax.ShapeDtypeStruct((2, N, FW), jnp.float32),
        mesh=mesh,
        scratch_types=[
            pltpu.VMEM((2, G, CHUNK), jnp.int32),
            pltpu.VMEM((2, G, CHUNK), jnp.int32),
            pltpu.VMEM((CHUNK, FW), jnp.float32),
            pltpu.VMEM((CHUNK, FW), jnp.float32),
            pltpu.VMEM((16, FW), jnp.float32),
            pltpu.VMEM_SHARED((RACC, FW), jnp.float32),
            pltpu.SemaphoreType.DMA,
            pltpu.SemaphoreType.DMA,
            pltpu.SemaphoreType.DMA,
            pltpu.SemaphoreType.DMA,
            pltpu.SemaphoreType.DMA,
        ],
        name=f"sc_spmm_k{kch}",
    )


def _mm_body(relu, split_out, post, h_ref, w_ref, b_ref, *rest):
    o_ref = rest[-1]
    res = (jnp.dot(h_ref[0], w_ref[0], preferred_element_type=jnp.float32)
           + jnp.dot(h_ref[1], w_ref[1], preferred_element_type=jnp.float32)
           + b_ref[...])
    if relu:
        res = jnp.maximum(res, 0.0)
    if post:
        wp_ref = rest[0]
        res = jnp.dot(res, wp_ref[...], preferred_element_type=jnp.float32)
        bm = res.shape[0]
        o_ref[...] = jnp.concatenate(
            [res, jnp.zeros((bm, FW - res.shape[1]), jnp.float32)], axis=1)
    elif split_out:
        o_ref[0] = res[:, :FW]
        o_ref[1] = res[:, FW:]
    else:
        o_ref[...] = res


def _mm2(a, wst, b, relu, split_out, w_post=None):
    fo = wst.shape[2]
    bm = 2000
    in_specs = [
        pl.BlockSpec((2, bm, FW), lambda i: (0, i, 0)),
        pl.BlockSpec((2, FW, fo), lambda i: (0, 0, 0)),
        pl.BlockSpec((1, fo), lambda i: (0, 0)),
    ]
    args = [a, wst, b.reshape(1, fo)]
    if w_post is not None:
        out_shape = jax.ShapeDtypeStruct((N, FW), jnp.float32)
        out_spec = pl.BlockSpec((bm, FW), lambda i: (i, 0))
        in_specs.append(pl.BlockSpec(w_post.shape, lambda i: (0, 0)))
        args.append(w_post)
    elif split_out:
        out_shape = jax.ShapeDtypeStruct((2, N, FW), jnp.float32)
        out_spec = pl.BlockSpec((2, bm, FW), lambda i: (0, i, 0))
    else:
        out_shape = jax.ShapeDtypeStruct((N, fo), jnp.float32)
        out_spec = pl.BlockSpec((bm, fo), lambda i: (i, 0))
    return pl.pallas_call(
        functools.partial(_mm_body, relu, split_out, w_post is not None),
        grid=(N // bm,),
        in_specs=in_specs,
        out_specs=out_spec,
        out_shape=out_shape,
    )(*args)


def _merge_body(fo, a_ref, b_ref, o_ref):
    o_ref[...] = (a_ref[0] + a_ref[1])[:, :fo] + b_ref[...]


def _merge(a, b4):
    fo = b4.shape[0]
    bm = 2000
    return pl.pallas_call(
        functools.partial(_merge_body, fo),
        grid=(N // bm,),
        in_specs=[
            pl.BlockSpec((2, bm, FW), lambda i: (0, i, 0)),
            pl.BlockSpec((1, fo), lambda i: (0, 0)),
        ],
        out_specs=pl.BlockSpec((bm, fo), lambda i: (i, 0)),
        out_shape=jax.ShapeDtypeStruct((N, fo), jnp.float32),
    )(a, b4.reshape(1, fo))


def kernel(x, edge_index, W0, W1, W2, W3, W4, b0, b1, b2, b3, b4):
    dst = edge_index[0].astype(jnp.int32)
    src = edge_index[1].astype(jnp.int32)
    pad = EPAD - E
    pad_idx = jnp.arange(pad, dtype=jnp.int32)
    dst_p = jnp.concatenate([dst, N + pad_idx % (RACC - N)])
    src_p = jnp.concatenate([src, pad_idx % N])
    src_r = src_p.reshape(NTILES, 160, CHUNK)
    dst_r = dst_p.reshape(NTILES, 160, CHUNK)
    src_fs = jnp.stack([src_r, src_r + N])
    dst_fs = jnp.stack([dst_r, dst_r])
    src_es = src_p.reshape(2, NTILES, 80, CHUNK)
    dst_es = dst_p.reshape(2, NTILES, 80, CHUNK)

    z_init = jnp.zeros((2, 16, FW), jnp.float32)
    spmm_fs = _make_spmm(160)
    spmm_es = _make_spmm(80)

    a = spmm_es(x, src_es, dst_es, z_init)
    h = _mm2(a, jnp.stack([W0, W0]), b0, relu=True, split_out=True)
    for W, b in ((W1, b1), (W2, b2)):
        a = spmm_fs(h.reshape(2 * N, FW), src_fs, dst_fs, z_init)
        h = _mm2(a, jnp.stack([W[:FW], W[FW:]]), b, relu=True, split_out=True)
    a = spmm_fs(h.reshape(2 * N, FW), src_fs, dst_fs, z_init)
    p = _mm2(a, jnp.stack([W3[:FW], W3[FW:]]), b3, relu=True, split_out=False,
             w_post=W4)
    a = spmm_es(p, src_es, dst_es, z_init)
    return _merge(a, b4)
